# Initial kernel scaffold; baseline (speedup 1.0000x reference)
import jax
import jax.numpy as jnp
from jax.experimental import pallas as pl


def kernel(sp_adj_idx, tp_adj_idx, sp_adj_val, tp_adj_val, sp_feat, tp_feat, label_idx, params):
    raise NotImplementedError("write your pallas kernel here")


# speedup vs baseline: 2.6598x; 2.6598x over previous
import functools

import jax
import jax.numpy as jnp
from jax import lax
from jax.experimental import pallas as pl
from jax.experimental.pallas import tpu as pltpu
from jax.experimental.pallas import tpu_sc as plsc

N = 10000
E = 320000
D = 128
EPS = 1e-3

NC = 2
NS = 16
L = 16
K = 80

NP = 10240
LB = 10240
BPW = LB // (NC * NS)

_GATHER_DNUMS = lax.GatherDimensionNumbers(
    offset_dims=(), collapsed_slice_dims=(0,), start_index_map=(0,))


def _splat(v16, e):
    idx = jnp.full((L, 1), e, jnp.int32)
    return lax.gather(v16, idx, _GATHER_DNUMS, (1,),
                      mode=lax.GatherScatterMode.PROMISE_IN_BOUNDS)


def _spmm_layer(sp_dst, sp_src, sp_val, tp_dst, tp_src, tp_val, x_sp, x_tp,
                zeros):
    mesh = plsc.VectorSubcoreMesh(core_axis_name="c", subcore_axis_name="s")

    @functools.partial(
        pl.kernel,
        out_type=(jax.ShapeDtypeStruct((NP, D), jnp.float32),
                  jax.ShapeDtypeStruct((NP, D), jnp.float32)),
        mesh=mesh,
        scratch_types=[
            pltpu.VMEM((K,), jnp.int32),
            pltpu.VMEM((K,), jnp.int32),
            pltpu.VMEM((K,), jnp.float32),
            pltpu.VMEM((K, D), jnp.float32),
            pltpu.VMEM_SHARED((NP, D), jnp.float32),
        ],
    )
    def k(sp_dst_h, sp_src_h, sp_val_h, tp_dst_h, tp_src_h, tp_val_h,
          xsp_h, xtp_h, z_h, osp_h, otp_h,
          src_v, dst_v, val_v, rows_v, agg_sh):
        c = lax.axis_index("c")
        s = lax.axis_index("s")
        rows_per_sub = NP // NS
        rslc = pl.ds(s * rows_per_sub, rows_per_sub)
        epw = E // NS

        def stream_body(dst_h, src_h, val_h, x_h, out_h):
            pltpu.sync_copy(z_h.at[rslc], agg_sh.at[rslc])
            plsc.subcore_barrier()

            base0 = s * epw

            @pl.loop(0, epw, step=K)
            def _(off):
                b = base0 + off
                pltpu.sync_copy(src_h.at[pl.ds(b, K)], src_v)
                pltpu.sync_copy(dst_h.at[pl.ds(b, K)], dst_v)
                pltpu.sync_copy(val_h.at[pl.ds(b, K)], val_v)
                pltpu.sync_copy(x_h.at[src_v], rows_v)

                @pl.loop(0, K, step=L)
                def _(j):
                    v16 = val_v[pl.ds(j, L)]
                    for e in range(L):
                        sv = _splat(v16, e)
                        r = rows_v.at[j + e]
                        for cc in range(D // L):
                            sl = pl.ds(cc * L, L)
                            r[sl] = r[sl] * sv

                pltpu.sync_copy(rows_v, agg_sh.at[dst_v], add=True)

            plsc.subcore_barrier()
            pltpu.sync_copy(agg_sh.at[rslc], out_h.at[rslc])

        @pl.when(c == 0)
        def _():
            stream_body(sp_dst_h, sp_src_h, sp_val_h, xsp_h, osp_h)

        @pl.when(c == 1)
        def _():
            stream_body(tp_dst_h, tp_src_h, tp_val_h, xtp_h, otp_h)

    return k(sp_dst, sp_src, sp_val, tp_dst, tp_src, tp_val, x_sp, x_tp,
             zeros)


def _dense_body(agg_ref, x_ref, w_ref, g_ref, b_ref, o_ref):
    h = jnp.dot(agg_ref[...], w_ref[...], preferred_element_type=jnp.float32)
    mu = jnp.mean(h, axis=0, keepdims=True)
    var = jnp.mean((h - mu) ** 2, axis=0, keepdims=True)
    hn = (h - mu) * lax.rsqrt(var + EPS) * g_ref[...] + b_ref[...]
    o_ref[...] = jnp.maximum(hn + x_ref[...], 0.0)


def _dense_layer(agg, x, w, g, b):
    return pl.pallas_call(
        _dense_body,
        out_shape=jax.ShapeDtypeStruct((N, D), jnp.float32),
    )(agg, x, w, g.reshape(1, D), b.reshape(1, D))


def _label_gather(s2, t2, idx_padded):
    mesh = plsc.VectorSubcoreMesh(core_axis_name="c", subcore_axis_name="s")

    @functools.partial(
        pl.kernel,
        out_type=(jax.ShapeDtypeStruct((LB, D), jnp.float32),
                  jax.ShapeDtypeStruct((LB, D), jnp.float32)),
        mesh=mesh,
        scratch_types=[
            pltpu.VMEM((BPW,), jnp.int32),
            pltpu.VMEM((BPW, D), jnp.float32),
        ],
    )
    def k(s_h, t_h, idx_h, os_h, ot_h, idx_v, rows_v):
        wid = lax.axis_index("s") * NC + lax.axis_index("c")
        bslc = pl.ds(wid * BPW, BPW)
        pltpu.sync_copy(idx_h.at[bslc], idx_v)
        pltpu.sync_copy(s_h.at[idx_v], rows_v)
        pltpu.sync_copy(rows_v, os_h.at[bslc])
        pltpu.sync_copy(t_h.at[idx_v], rows_v)
        pltpu.sync_copy(rows_v, ot_h.at[bslc])

    return k(s2, t2, idx_padded)


def _head_body(sg_ref, tg_ref, sw1_ref, sb1_ref, sw2_ref, sb2_ref,
               tw1_ref, tb1_ref, tw2_ref, tb2_ref,
               c1a_ref, c1b_ref, cb1_ref, c2_ref, cb2_ref, c3_ref, cb3_ref,
               o_ref):
    dot = functools.partial(jnp.dot, preferred_element_type=jnp.float32)
    sg = sg_ref[...]
    tg = tg_ref[...]
    sp = dot(jnp.maximum(dot(sg, sw1_ref[...]) + sb1_ref[...], 0.0),
             sw2_ref[...]) + sb2_ref[...]
    tp_ = dot(jnp.maximum(dot(tg, tw1_ref[...]) + tb1_ref[...], 0.0),
              tw2_ref[...]) + tb2_ref[...]
    z1 = jnp.maximum(dot(sp, c1a_ref[...]) + dot(tp_, c1b_ref[...])
                     + cb1_ref[...], 0.0)
    z2 = jnp.maximum(dot(z1, c2_ref[...]) + cb2_ref[...], 0.0)
    o_ref[...] = dot(z2, c3_ref[...]) + cb3_ref[...]


def _head(sg, tg, p):
    blk = 2000
    grid = (N // blk,)
    row_spec = pl.BlockSpec((blk, D), lambda i: (i, 0))

    def w_spec(a):
        return pl.BlockSpec(a.shape, lambda i: tuple(0 for _ in a.shape))

    c1a = p["c_W1"][:D]
    c1b = p["c_W1"][D:]
    weights = [p["spc_W1"], p["spc_b1"].reshape(1, -1),
               p["spc_W2"], p["spc_b2"].reshape(1, -1),
               p["tpc_W1"], p["tpc_b1"].reshape(1, -1),
               p["tpc_W2"], p["tpc_b2"].reshape(1, -1),
               c1a, c1b, p["c_b1"].reshape(1, -1),
               p["c_W2"], p["c_b2"].reshape(1, -1),
               p["c_W3"], p["c_b3"].reshape(1, -1)]
    return pl.pallas_call(
        _head_body,
        grid=grid,
        in_specs=[row_spec, row_spec] + [w_spec(w) for w in weights],
        out_specs=pl.BlockSpec((blk, 2), lambda i: (i, 0)),
        out_shape=jax.ShapeDtypeStruct((N, 2), jnp.float32),
    )(sg, tg, *weights)


def kernel(sp_adj_idx, tp_adj_idx, sp_adj_val, tp_adj_val, sp_feat, tp_feat,
           label_idx, params):
    p = params
    sp_dst = sp_adj_idx[:, 0]
    sp_src = sp_adj_idx[:, 1]
    tp_dst = tp_adj_idx[:, 0]
    tp_src = tp_adj_idx[:, 1]
    zeros = jnp.zeros((NP, D), jnp.float32)

    agg_s1, agg_t1 = _spmm_layer(sp_dst, sp_src, sp_adj_val,
                                 tp_dst, tp_src, tp_adj_val,
                                 sp_feat, tp_feat, zeros)
    s1 = _dense_layer(agg_s1[:N], sp_feat, p["W_sp1"], p["g_sp1"], p["b_sp1"])
    t1 = _dense_layer(agg_t1[:N], tp_feat, p["W_tp1"], p["g_tp1"], p["b_tp1"])

    agg_s2, agg_t2 = _spmm_layer(sp_dst, sp_src, sp_adj_val,
                                 tp_dst, tp_src, tp_adj_val,
                                 s1, t1, zeros)
    s2 = _dense_layer(agg_s2[:N], s1, p["W_sp2"], p["g_sp2"], p["b_sp2"])
    t2 = _dense_layer(agg_t2[:N], t1, p["W_tp2"], p["g_tp2"], p["b_tp2"])

    idx_padded = jnp.concatenate(
        [label_idx.astype(jnp.int32),
         jnp.zeros((LB - N,), jnp.int32)])
    sg_p, tg_p = _label_gather(s2, t2, idx_padded)
    return _head(sg_p[:N], tg_p[:N], p)


# speedup vs baseline: 3.7517x; 1.4106x over previous
import dataclasses
import functools

import jax
import jax.numpy as jnp
from jax import lax
from jax.experimental import pallas as pl
from jax.experimental.pallas import tpu as pltpu
from jax.experimental.pallas import tpu_sc as plsc

N = 10000
E = 320000
D = 128
EPS = 1e-3

NC = 2
NS = 16
L = 16
K = 48

NP = 10240
LB = 10240
BPW = LB // (NC * NS)

_GATHER_DNUMS = lax.GatherDimensionNumbers(
    offset_dims=(), collapsed_slice_dims=(0,), start_index_map=(0,))


def _splat(v16, e):
    idx = jnp.full((L, 1), e, jnp.int32)
    return lax.gather(v16, idx, _GATHER_DNUMS, (1,),
                      mode=lax.GatherScatterMode.PROMISE_IN_BOUNDS)


CHP = 420
EPWP = CHP * K

_SC_PARAMS = pltpu.CompilerParams()
if "needs_layout_passes" in pltpu.CompilerParams.__dataclass_fields__:
    _SC_PARAMS = dataclasses.replace(_SC_PARAMS, needs_layout_passes=False)


def _spmm_layer(sp_src, sp_dst, sp_val, tp_src, tp_dst, tp_val, x_sp, x_tp,
                zeros):
    mesh = plsc.VectorSubcoreMesh(core_axis_name="c", subcore_axis_name="s")

    idx_slots = [pltpu.VMEM((K,), jnp.int32) for _ in range(8)]
    val_slots = [pltpu.VMEM((K,), jnp.float32) for _ in range(4)]
    row_slots = [pltpu.VMEM((K, D), jnp.float32) for _ in range(4)]
    sems = [pltpu.SemaphoreType.DMA for _ in range(8)]

    @functools.partial(
        pl.kernel,
        out_type=(jax.ShapeDtypeStruct((NP, D), jnp.float32),
                  jax.ShapeDtypeStruct((NP, D), jnp.float32)),
        mesh=mesh,
        scratch_types=idx_slots + val_slots + row_slots
        + [pltpu.VMEM_SHARED((NP, D), jnp.float32)] + sems,
    )
    def k(sp_src_h, sp_dst_h, sp_val_h, tp_src_h, tp_dst_h, tp_val_h,
          xsp_h, xtp_h, z_h, osp_h, otp_h,
          src0, src1, src2, src3, dst0, dst1, dst2, dst3,
          val0, val1, val2, val3, rows0, rows1, sc0, sc1, agg_sh,
          isem0, isem1, isem2, isem3, gsem0, gsem1, ssem0, ssem1):
        c = lax.axis_index("c")
        s = lax.axis_index("s")
        rows_per_sub = NP // NS
        rslc = pl.ds(s * rows_per_sub, rows_per_sub)
        src_q = (src0, src1, src2, src3)
        dst_q = (dst0, dst1, dst2, dst3)
        val_q = (val0, val1, val2, val3)
        isem_q = (isem0, isem1, isem2, isem3)
        rows_b = (rows0, rows1)
        sc_b = (sc0, sc1)
        gsem_b = (gsem0, gsem1)
        ssem_b = (ssem0, ssem1)

        def stream_body(src_h, dst_h, val_h, x_h, out_h):
            pltpu.sync_copy(z_h.at[rslc], agg_sh.at[rslc])
            plsc.subcore_barrier()

            base = s * EPWP

            def idx3(n, q):
                slc = pl.ds(base + n * K, K)
                return (pltpu.make_async_copy(src_h.at[slc], src_q[q],
                                              isem_q[q]),
                        pltpu.make_async_copy(dst_h.at[slc], dst_q[q],
                                              isem_q[q]),
                        pltpu.make_async_copy(val_h.at[slc], val_q[q],
                                              isem_q[q]))

            def idx_start(n, q):
                for d in idx3(n, q):
                    d.start()

            def idx_wait(n, q):
                for d in idx3(n, q):
                    d.wait()

            def gather(n, b, q):
                del n
                return pltpu.make_async_copy(
                    x_h.at[src_q[q]], rows_b[b], gsem_b[b])

            def scatter(n, b, q):
                del n
                return pltpu.make_async_copy(
                    sc_b[b], agg_sh.at[dst_q[q]], ssem_b[b])

            idx_start(0, 0)
            idx_start(1, 1)
            idx_wait(0, 0)
            gather(0, 0, 0).start()
            idx_wait(1, 1)
            gather(1, 1, 1).start()
            idx_start(2, 2)
            idx_start(3, 3)

            @pl.loop(0, CHP, step=4)
            def _(g):
                for k4 in range(4):
                    n = g + k4
                    b = k4 % 2
                    q = k4
                    q2 = (k4 + 2) % 4
                    gather(n, b, q).wait()

                    @pl.when(n >= 1)
                    def _():
                        scatter(n - 1, 1 - b, (k4 + 3) % 4).wait()

                    @pl.when(jnp.logical_and(n >= 2, n + 2 < CHP))
                    def _():
                        idx_start(n + 2, q2)

                    @pl.loop(0, K, step=L)
                    def _(j):
                        v16 = val_q[q][pl.ds(j, L)]
                        for e in range(L):
                            sv = _splat(v16, e)
                            rsrc = rows_b[b].at[j + e]
                            rdst = sc_b[b].at[j + e]
                            for cc in range(D // L):
                                sl = pl.ds(cc * L, L)
                                rdst[sl] = rsrc[sl] * sv

                    @pl.when(n + 2 < CHP)
                    def _():
                        idx_wait(n + 2, q2)
                        gather(n + 2, b, q2).start()

                    scatter(n, b, q).start(add=True)

            scatter(CHP - 1, 1, (CHP - 1) % 4).wait()
            plsc.subcore_barrier()
            pltpu.sync_copy(agg_sh.at[rslc], out_h.at[rslc])

        @pl.when(c == 0)
        def _():
            stream_body(sp_src_h, sp_dst_h, sp_val_h, xsp_h, osp_h)

        @pl.when(c == 1)
        def _():
            stream_body(tp_src_h, tp_dst_h, tp_val_h, xtp_h, otp_h)

    return k(sp_src, sp_dst, sp_val, tp_src, tp_dst, tp_val, x_sp, x_tp,
             zeros)


def _dense_body(agg_ref, x_ref, w_ref, g_ref, b_ref, o_ref):
    h = jnp.dot(agg_ref[...], w_ref[...], preferred_element_type=jnp.float32)
    mu = jnp.mean(h, axis=0, keepdims=True)
    var = jnp.mean((h - mu) ** 2, axis=0, keepdims=True)
    hn = (h - mu) * lax.rsqrt(var + EPS) * g_ref[...] + b_ref[...]
    o_ref[...] = jnp.maximum(hn + x_ref[...], 0.0)


def _dense_layer(agg, x, w, g, b):
    return pl.pallas_call(
        _dense_body,
        out_shape=jax.ShapeDtypeStruct((N, D), jnp.float32),
    )(agg, x, w, g.reshape(1, D), b.reshape(1, D))


def _label_gather(s2, t2, idx_padded):
    mesh = plsc.VectorSubcoreMesh(core_axis_name="c", subcore_axis_name="s")

    @functools.partial(
        pl.kernel,
        out_type=(jax.ShapeDtypeStruct((LB, D), jnp.float32),
                  jax.ShapeDtypeStruct((LB, D), jnp.float32)),
        mesh=mesh,
        scratch_types=[
            pltpu.VMEM((BPW,), jnp.int32),
            pltpu.VMEM((BPW, D), jnp.float32),
        ],
    )
    def k(s_h, t_h, idx_h, os_h, ot_h, idx_v, rows_v):
        wid = lax.axis_index("s") * NC + lax.axis_index("c")
        bslc = pl.ds(wid * BPW, BPW)
        pltpu.sync_copy(idx_h.at[bslc], idx_v)
        pltpu.sync_copy(s_h.at[idx_v], rows_v)
        pltpu.sync_copy(rows_v, os_h.at[bslc])
        pltpu.sync_copy(t_h.at[idx_v], rows_v)
        pltpu.sync_copy(rows_v, ot_h.at[bslc])

    return k(s2, t2, idx_padded)


def _head_body(sg_ref, tg_ref, sw1_ref, sb1_ref, sw2_ref, sb2_ref,
               tw1_ref, tb1_ref, tw2_ref, tb2_ref,
               c1a_ref, c1b_ref, cb1_ref, c2_ref, cb2_ref, c3_ref, cb3_ref,
               o_ref):
    dot = functools.partial(jnp.dot, preferred_element_type=jnp.float32)
    sg = sg_ref[...]
    tg = tg_ref[...]
    sp = dot(jnp.maximum(dot(sg, sw1_ref[...]) + sb1_ref[...], 0.0),
             sw2_ref[...]) + sb2_ref[...]
    tp_ = dot(jnp.maximum(dot(tg, tw1_ref[...]) + tb1_ref[...], 0.0),
              tw2_ref[...]) + tb2_ref[...]
    z1 = jnp.maximum(dot(sp, c1a_ref[...]) + dot(tp_, c1b_ref[...])
                     + cb1_ref[...], 0.0)
    z2 = jnp.maximum(dot(z1, c2_ref[...]) + cb2_ref[...], 0.0)
    o_ref[...] = dot(z2, c3_ref[...]) + cb3_ref[...]


def _head(sg, tg, p):
    blk = 2000
    grid = (N // blk,)
    row_spec = pl.BlockSpec((blk, D), lambda i: (i, 0))

    def w_spec(a):
        return pl.BlockSpec(a.shape, lambda i: tuple(0 for _ in a.shape))

    c1a = p["c_W1"][:D]
    c1b = p["c_W1"][D:]
    weights = [p["spc_W1"], p["spc_b1"].reshape(1, -1),
               p["spc_W2"], p["spc_b2"].reshape(1, -1),
               p["tpc_W1"], p["tpc_b1"].reshape(1, -1),
               p["tpc_W2"], p["tpc_b2"].reshape(1, -1),
               c1a, c1b, p["c_b1"].reshape(1, -1),
               p["c_W2"], p["c_b2"].reshape(1, -1),
               p["c_W3"], p["c_b3"].reshape(1, -1)]
    return pl.pallas_call(
        _head_body,
        grid=grid,
        in_specs=[row_spec, row_spec] + [w_spec(w) for w in weights],
        out_specs=pl.BlockSpec((blk, 2), lambda i: (i, 0)),
        out_shape=jax.ShapeDtypeStruct((N, 2), jnp.float32),
    )(sg, tg, *weights)


def kernel(sp_adj_idx, tp_adj_idx, sp_adj_val, tp_adj_val, sp_feat, tp_feat,
           label_idx, params):
    p = params

    def pad_edges(a):
        a2 = a.reshape(NS, E // NS)
        a2 = jnp.pad(a2, ((0, 0), (0, EPWP - E // NS)))
        return a2.reshape(NS * EPWP)

    sp_src = pad_edges(sp_adj_idx[:, 1])
    sp_dst = pad_edges(sp_adj_idx[:, 0])
    sp_val = pad_edges(sp_adj_val)
    tp_src = pad_edges(tp_adj_idx[:, 1])
    tp_dst = pad_edges(tp_adj_idx[:, 0])
    tp_val = pad_edges(tp_adj_val)
    zeros = jnp.zeros((NP, D), jnp.float32)

    agg_s1, agg_t1 = _spmm_layer(sp_src, sp_dst, sp_val, tp_src, tp_dst,
                                 tp_val, sp_feat, tp_feat, zeros)
    s1 = _dense_layer(agg_s1[:N], sp_feat, p["W_sp1"], p["g_sp1"], p["b_sp1"])
    t1 = _dense_layer(agg_t1[:N], tp_feat, p["W_tp1"], p["g_tp1"], p["b_tp1"])

    agg_s2, agg_t2 = _spmm_layer(sp_src, sp_dst, sp_val, tp_src, tp_dst,
                                 tp_val, s1, t1, zeros)
    s2 = _dense_layer(agg_s2[:N], s1, p["W_sp2"], p["g_sp2"], p["b_sp2"])
    t2 = _dense_layer(agg_t2[:N], t1, p["W_tp2"], p["g_tp2"], p["b_tp2"])

    idx_padded = jnp.concatenate(
        [label_idx.astype(jnp.int32),
         jnp.zeros((LB - N,), jnp.int32)])
    sg_p, tg_p = _label_gather(s2, t2, idx_padded)
    return _head(sg_p[:N], tg_p[:N], p)
